# Initial kernel scaffold; baseline (speedup 1.0000x reference)
import jax
import jax.numpy as jnp
from jax.experimental import pallas as pl


def kernel(x, wte, wpe):
    raise NotImplementedError("write your pallas kernel here")


# speedup vs baseline: 1.6721x; 1.6721x over previous
import functools

import jax
import jax.numpy as jnp
from jax import lax
from jax.experimental import pallas as pl
from jax.experimental.pallas import tpu as pltpu
from jax.experimental.pallas import tpu_sc as plsc

N_VOCAB = 100000
N_CTX = 2048
N_EMBED = 128
BATCH = 4

NC = 2
NS = 16
NW = NC * NS
NTOK = BATCH * N_CTX
BPW = NTOK // NW
GCH = 128
NG = BPW // GCH


def _sc_embed(x_hbm, wte_hbm, wpe_hbm, out_hbm, idx_v, rows_v, pev_v, sem_g, sem_p):
    wid = lax.axis_index("s") * NC + lax.axis_index("c")
    base = wid * BPW
    t0 = lax.rem(base, N_CTX)

    pltpu.sync_copy(x_hbm.at[pl.ds(wid * NG, NG)], idx_v)

    cps = [
        pltpu.async_copy(
            wte_hbm.at[idx_v.at[g]], rows_v.at[pl.ds(g * GCH, GCH)], sem_g
        )
        for g in range(NG)
    ]
    cp_p = pltpu.async_copy(wpe_hbm.at[pl.ds(t0, BPW)], pev_v, sem_p)
    for cp in cps:
        cp.wait()
    cp_p.wait()

    @pl.loop(0, BPW)
    def _(r):
        for c in range(N_EMBED // 16):
            sl = pl.ds(c * 16, 16)
            plsc.addupdate(rows_v.at[r, sl], pev_v[r, sl])

    pltpu.sync_copy(rows_v, out_hbm.at[pl.ds(base, BPW)])


@jax.jit
def _embed(x2d, wte, wpe):
    mesh = plsc.VectorSubcoreMesh(core_axis_name="c", subcore_axis_name="s")
    run = functools.partial(
        pl.kernel,
        out_type=jax.ShapeDtypeStruct((NTOK, N_EMBED), jnp.float32),
        mesh=mesh,
        scratch_types=[
            pltpu.VMEM((NG, GCH), jnp.int32),
            pltpu.VMEM((BPW, N_EMBED), jnp.float32),
            pltpu.VMEM((BPW, N_EMBED), jnp.float32),
            pltpu.SemaphoreType.DMA,
            pltpu.SemaphoreType.DMA,
        ],
    )(_sc_embed)
    return run(x2d, wte, wpe)


def kernel(x, wte, wpe):
    x2d = x.astype(jnp.int32).reshape(NTOK // GCH, GCH)
    out = _embed(x2d, wte, wpe)
    return out.reshape(BATCH, N_CTX, N_EMBED)


# speedup vs baseline: 1.7546x; 1.0493x over previous
import functools

import jax
import jax.numpy as jnp
from jax import lax
from jax.experimental import pallas as pl
from jax.experimental.pallas import tpu as pltpu
from jax.experimental.pallas import tpu_sc as plsc

N_VOCAB = 100000
N_CTX = 2048
N_EMBED = 128
BATCH = 4

NC = 2
NS = 16
NW = NC * NS
NTOK = BATCH * N_CTX
BPW = NTOK // NW
GCH = 128
NG = BPW // GCH


def _sc_embed(x_hbm, wte_hbm, wpe_hbm, out_hbm, idx_v, rows_v, sem_g):
    wid = lax.axis_index("s") * NC + lax.axis_index("c")
    base = wid * BPW
    t0 = lax.rem(base, N_CTX)

    pltpu.sync_copy(x_hbm.at[pl.ds(wid * NG, NG)], idx_v)

    pltpu.sync_copy(wpe_hbm.at[pl.ds(t0, BPW)], rows_v)
    cps = [
        pltpu.async_copy(
            wte_hbm.at[idx_v.at[g]],
            rows_v.at[pl.ds(g * GCH, GCH)],
            sem_g,
            add=True,
        )
        for g in range(NG)
    ]
    for cp in cps:
        cp.wait()

    pltpu.sync_copy(rows_v, out_hbm.at[pl.ds(base, BPW)])


@jax.jit
def _embed(x2d, wte, wpe):
    mesh = plsc.VectorSubcoreMesh(core_axis_name="c", subcore_axis_name="s")
    run = functools.partial(
        pl.kernel,
        out_type=jax.ShapeDtypeStruct((NTOK, N_EMBED), jnp.float32),
        mesh=mesh,
        scratch_types=[
            pltpu.VMEM((NG, GCH), jnp.int32),
            pltpu.VMEM((BPW, N_EMBED), jnp.float32),
            pltpu.SemaphoreType.DMA,
        ],
    )(_sc_embed)
    return run(x2d, wte, wpe)


def kernel(x, wte, wpe):
    x2d = x.astype(jnp.int32).reshape(NTOK // GCH, GCH)
    out = _embed(x2d, wte, wpe)
    return out.reshape(BATCH, N_CTX, N_EMBED)


# speedup vs baseline: 1.8189x; 1.0367x over previous
import functools

import jax
import jax.numpy as jnp
from jax import lax
from jax.experimental import pallas as pl
from jax.experimental.pallas import tpu as pltpu
from jax.experimental.pallas import tpu_sc as plsc

N_VOCAB = 100000
N_CTX = 2048
N_EMBED = 128
BATCH = 4

NC = 2
NS = 16
NW = NC * NS
NTOK = BATCH * N_CTX
BPW = NTOK // NW
WPB = N_CTX // BPW
GCH = 64
NG = BPW // GCH


def _sc_embed(x_hbm, wte_hbm, wpe_hbm, out_hbm, idx_v, rows_v, sem_i, sem_p, sem_g, sem_s):
    wid = lax.axis_index("s") * NC + lax.axis_index("c")
    b = wid // WPB
    t0 = lax.rem(wid, WPB) * BPW

    cp_i = [
        pltpu.async_copy(x_hbm.at[b, pl.ds(t0 + g * GCH, GCH)], idx_v.at[g], sem_i)
        for g in range(NG)
    ]
    cp_p = [
        pltpu.async_copy(
            wpe_hbm.at[pl.ds(t0 + g * GCH, GCH)],
            rows_v.at[pl.ds(g * GCH, GCH)],
            sem_p,
        )
        for g in range(NG)
    ]
    cp_g = []
    for g in range(NG):
        cp_i[g].wait()
        cp_p[g].wait()
        cp_g.append(
            pltpu.async_copy(
                wte_hbm.at[idx_v.at[g]],
                rows_v.at[pl.ds(g * GCH, GCH)],
                sem_g,
                add=True,
            )
        )
    cp_s = []
    for g in range(NG):
        cp_g[g].wait()
        cp_s.append(
            pltpu.async_copy(
                rows_v.at[pl.ds(g * GCH, GCH)],
                out_hbm.at[b, pl.ds(t0 + g * GCH, GCH)],
                sem_s,
            )
        )
    for cp in cp_s:
        cp.wait()


@jax.jit
def _embed(x, wte, wpe):
    mesh = plsc.VectorSubcoreMesh(core_axis_name="c", subcore_axis_name="s")
    run = functools.partial(
        pl.kernel,
        out_type=jax.ShapeDtypeStruct((BATCH, N_CTX, N_EMBED), jnp.float32),
        mesh=mesh,
        scratch_types=[
            pltpu.VMEM((NG, GCH), jnp.int32),
            pltpu.VMEM((BPW, N_EMBED), jnp.float32),
            pltpu.SemaphoreType.DMA,
            pltpu.SemaphoreType.DMA,
            pltpu.SemaphoreType.DMA,
            pltpu.SemaphoreType.DMA,
        ],
    )(_sc_embed)
    return run(x, wte, wpe)


def kernel(x, wte, wpe):
    return _embed(x.astype(jnp.int32), wte, wpe)
